# baseline (device time: 272399 ns/iter reference)
import jax
import jax.numpy as jnp
from jax import lax
from jax.experimental import pallas as pl
from jax.experimental.pallas import tpu as pltpu

N_DEV = 8
SQ_BLK = 256
RH = 128
D_MODEL = 1024
HEADS = 8
DH = 128
SKV = 4096
WIN = 384
SCALE = 0.08838834764831843


def kernel(x, Wq, K_ext, V_ext, Wo):
    d = lax.axis_index("i")
    hs = d * HEADS

    x_bf = x[0].astype(jnp.bfloat16)
    wq_bf = Wq.astype(jnp.bfloat16)
    wo_bf = Wo.astype(jnp.bfloat16)
    k2 = K_ext.reshape(SKV, 64 * DH)
    v2 = V_ext.reshape(SKV, 64 * DH)

    def body(x_ref, wq_ref, k_hbm, v_hbm, wo_ref, out_ref,
             xf_r, xf_l, pa_r, pa_l, rr_r, rr_l,
             kw_r, vw_r, kw_l, vw_l,
             ag_ss_r, ag_rs_r, rs_ss_r, rs_rs_r,
             ag_ss_l, ag_rs_l, rs_ss_l, rs_rs_l,
             kd_r, vd_r, kd_l, vd_l):
        my = lax.axis_index("i")
        left = lax.rem(my + N_DEV - 1, N_DEV)
        right = lax.rem(my + 1, N_DEV)

        barrier = pltpu.get_barrier_semaphore()
        for nbr in (left, right):
            pl.semaphore_signal(barrier, inc=1, device_id=(nbr,),
                                device_id_type=pl.DeviceIdType.MESH)
        pl.semaphore_wait(barrier, 2)

        xf_r[pl.ds(my, 1)] = x_ref[:RH][None]
        xf_l[pl.ds(my, 1)] = x_ref[RH:][None]

        rings = (
            (+1, 0, xf_r, pa_r, rr_r, ag_ss_r, ag_rs_r, rs_ss_r, rs_rs_r,
             right),
            (-1, RH, xf_l, pa_l, rr_l, ag_ss_l, ag_rs_l, rs_ss_l, rs_rs_l,
             left),
        )

        def win_start(b, off):
            s0 = jnp.clip(b * SQ_BLK + off - 128, 0, SKV - WIN)
            return pl.multiple_of(s0, 128)

        wins = ((kw_r, vw_r, kd_r, vd_r), (kw_l, vw_l, kd_l, vd_l))

        def win_dmas(ri, h):
            dr, off = rings[ri][0], rings[ri][1]
            kw, vw, kd, vd = wins[ri]
            b = lax.rem(my - dr * h + 2 * N_DEV, N_DEV)
            st = win_start(b, off)
            col = my * HEADS * DH
            slot = h % 2
            return (
                pltpu.make_async_copy(
                    k_hbm.at[pl.ds(st, WIN), pl.ds(col, HEADS * DH)],
                    kw.at[slot], kd.at[slot]),
                pltpu.make_async_copy(
                    v_hbm.at[pl.ds(st, WIN), pl.ds(col, HEADS * DH)],
                    vw.at[slot], vd.at[slot]),
            )

        def compute_half(b, off, xf, kb, vb, start):
            xb = xf[pl.ds(b, 1)][0]
            q = jnp.dot(xb, wq_ref[...],
                        preferred_element_type=jnp.float32)
            qb = (q * SCALE).astype(jnp.bfloat16)
            ctx_parts = []
            for hh in range(HEADS):
                qh = qb[:, hh * DH:(hh + 1) * DH]
                kh = kb[:, hh * DH:(hh + 1) * DH]
                s = lax.dot_general(qh, kh, (((1,), (1,)), ((), ())),
                                    preferred_element_type=jnp.float32)
                qi = b * SQ_BLK + off + lax.broadcasted_iota(
                    jnp.int32, (RH, WIN), 0)
                ki = start + lax.broadcasted_iota(jnp.int32, (RH, WIN), 1)
                s = jnp.where(jnp.abs(qi - ki) <= 128, s, -1e9)
                m = jnp.max(s, axis=1, keepdims=True)
                w = jnp.exp(s - m)
                w = w / jnp.sum(w, axis=1, keepdims=True)
                vh = vb[:, hh * DH:(hh + 1) * DH]
                ctx_parts.append(jnp.dot(w.astype(jnp.bfloat16), vh,
                                         preferred_element_type=jnp.float32))
            ctx = jnp.concatenate(ctx_parts, axis=1).astype(jnp.bfloat16)
            partial = jnp.dot(ctx, wo_ref[...],
                              preferred_element_type=jnp.float32)
            return partial.astype(jnp.bfloat16)

        ag_sends = []
        rs_sends = []

        def rs_step(ring, s):
            dr, off, xf, pa, rr, ag_ss, ag_rs, rs_ss, rs_rs, tgt = ring
            bs = lax.rem(my - dr * (s + 1) + 2 * N_DEV, N_DEV)
            if s >= 1:
                pltpu.make_async_remote_copy(
                    src_ref=rr.at[s - 1], dst_ref=rr.at[s - 1],
                    send_sem=rs_ss.at[s - 1], recv_sem=rs_rs.at[s - 1],
                    device_id=(tgt,), device_id_type=pl.DeviceIdType.MESH,
                ).wait_recv()
                acc = (pa[pl.ds(bs, 1)][0].astype(jnp.float32)
                       + rr[s - 1].astype(jnp.float32))
                pa[pl.ds(bs, 1)] = acc.astype(jnp.bfloat16)[None]
            rdma = pltpu.make_async_remote_copy(
                src_ref=pa.at[bs], dst_ref=rr.at[s],
                send_sem=rs_ss.at[s], recv_sem=rs_rs.at[s],
                device_id=(tgt,), device_id_type=pl.DeviceIdType.MESH,
            )
            rdma.start()
            rs_sends.append(rdma)

        for ri in range(2):
            for c in win_dmas(ri, 0):
                c.start()

        for h in range(N_DEV):
            if h < N_DEV - 1:
                for ring in rings:
                    dr, off, xf = ring[0], ring[1], ring[2]
                    ag_ss, ag_rs, tgt = ring[5], ring[6], ring[9]
                    b = lax.rem(my - dr * h + 2 * N_DEV, N_DEV)
                    rdma = pltpu.make_async_remote_copy(
                        src_ref=xf.at[b], dst_ref=xf.at[b],
                        send_sem=ag_ss.at[h], recv_sem=ag_rs.at[h],
                        device_id=(tgt,),
                        device_id_type=pl.DeviceIdType.MESH,
                    )
                    rdma.start()
                    ag_sends.append(rdma)

                for ri in range(2):
                    for c in win_dmas(ri, h + 1):
                        c.start()

            if h >= 2:
                for ring in rings:
                    rs_step(ring, h - 2)

            for ri, (dr, off, xf, pa, rr, ag_ss, ag_rs, rs_ss, rs_rs,
                     tgt) in enumerate(rings):
                b = lax.rem(my - dr * h + 2 * N_DEV, N_DEV)
                for c in win_dmas(ri, h):
                    c.wait()
                kw, vw = wins[ri][0], wins[ri][1]
                kb = kw[h % 2].astype(jnp.bfloat16)
                vb = vw[h % 2].astype(jnp.bfloat16)
                chunk = compute_half(b, off, xf, kb, vb, win_start(b, off))
                pa[pl.ds(b, 1)] = chunk[None]

            if h < N_DEV - 1:
                for (dr, off, xf, pa, rr, ag_ss, ag_rs, rs_ss, rs_rs,
                     tgt) in rings:
                    b_next = lax.rem(my - dr * (h + 1) + 2 * N_DEV, N_DEV)
                    pltpu.make_async_remote_copy(
                        src_ref=xf.at[b_next], dst_ref=xf.at[b_next],
                        send_sem=ag_ss.at[h], recv_sem=ag_rs.at[h],
                        device_id=(tgt,),
                        device_id_type=pl.DeviceIdType.MESH,
                    ).wait_recv()

        for ring in rings:
            rs_step(ring, N_DEV - 2)

        outs = []
        for dr, off, xf, pa, rr, ag_ss, ag_rs, rs_ss, rs_rs, tgt in rings:
            pltpu.make_async_remote_copy(
                src_ref=rr.at[N_DEV - 2], dst_ref=rr.at[N_DEV - 2],
                send_sem=rs_ss.at[N_DEV - 2], recv_sem=rs_rs.at[N_DEV - 2],
                device_id=(tgt,), device_id_type=pl.DeviceIdType.MESH,
            ).wait_recv()
            outs.append((pa[pl.ds(my, 1)][0].astype(jnp.float32)
                         + rr[N_DEV - 2].astype(jnp.float32)))
        out_ref[0] = jnp.concatenate(outs, axis=0)

        for rdma in ag_sends + rs_sends:
            rdma.wait_send()

    out = pl.pallas_call(
        body,
        out_shape=jax.ShapeDtypeStruct((1, SQ_BLK, D_MODEL), jnp.float32),
        in_specs=[
            pl.BlockSpec(memory_space=pltpu.VMEM),
            pl.BlockSpec(memory_space=pltpu.VMEM),
            pl.BlockSpec(memory_space=pl.ANY),
            pl.BlockSpec(memory_space=pl.ANY),
            pl.BlockSpec(memory_space=pltpu.VMEM),
        ],
        out_specs=pl.BlockSpec(memory_space=pltpu.VMEM),
        scratch_shapes=[
            pltpu.VMEM((N_DEV, RH, D_MODEL), jnp.bfloat16),
            pltpu.VMEM((N_DEV, RH, D_MODEL), jnp.bfloat16),
            pltpu.VMEM((N_DEV, RH, D_MODEL), jnp.bfloat16),
            pltpu.VMEM((N_DEV, RH, D_MODEL), jnp.bfloat16),
            pltpu.VMEM((N_DEV - 1, RH, D_MODEL), jnp.bfloat16),
            pltpu.VMEM((N_DEV - 1, RH, D_MODEL), jnp.bfloat16),
            pltpu.VMEM((2, WIN, HEADS * DH), jnp.float32),
            pltpu.VMEM((2, WIN, HEADS * DH), jnp.float32),
            pltpu.VMEM((2, WIN, HEADS * DH), jnp.float32),
            pltpu.VMEM((2, WIN, HEADS * DH), jnp.float32),
        ] + [pltpu.SemaphoreType.DMA((N_DEV - 1,))] * 8
          + [pltpu.SemaphoreType.DMA((2,))] * 4,
        compiler_params=pltpu.CompilerParams(collective_id=0),
    )(x_bf, wq_bf, k2, v2, wo_bf)
    return out


# device time: 80878 ns/iter; 3.3680x vs baseline; 3.3680x over previous
import jax
import jax.numpy as jnp
from jax import lax
from jax.experimental import pallas as pl
from jax.experimental.pallas import tpu as pltpu

N_DEV = 8
SQ_BLK = 256
RH = 128
D_MODEL = 1024
HEADS = 8
DH = 128
SKV = 4096
WIN = 384
SCALE = 0.08838834764831843


def kernel(x, Wq, K_ext, V_ext, Wo):
    d = lax.axis_index("i")
    hs = d * HEADS

    x_bf = x[0].astype(jnp.bfloat16)
    wq_bf = Wq.astype(jnp.bfloat16)
    wo_bf = Wo.astype(jnp.bfloat16)
    k2 = K_ext[0]
    v2 = V_ext[0]

    def body(x_ref, wq_ref, k_hbm, v_hbm, wo_ref, out_ref,
             xf_r, xf_l, pa_r, pa_l, rr_r, rr_l,
             kw_r, vw_r, kw_l, vw_l,
             ag_ss_r, ag_rs_r, rs_ss_r, rs_rs_r,
             ag_ss_l, ag_rs_l, rs_ss_l, rs_rs_l,
             kd_r, vd_r, kd_l, vd_l):
        my = lax.axis_index("i")
        left = lax.rem(my + N_DEV - 1, N_DEV)
        right = lax.rem(my + 1, N_DEV)

        barrier = pltpu.get_barrier_semaphore()
        for nbr in (left, right):
            pl.semaphore_signal(barrier, inc=1, device_id=(nbr,),
                                device_id_type=pl.DeviceIdType.MESH)
        pl.semaphore_wait(barrier, 2)

        xf_r[pl.ds(my, 1)] = x_ref[:RH][None]
        xf_l[pl.ds(my, 1)] = x_ref[RH:][None]

        rings = (
            (+1, 0, xf_r, pa_r, rr_r, ag_ss_r, ag_rs_r, rs_ss_r, rs_rs_r,
             right),
            (-1, RH, xf_l, pa_l, rr_l, ag_ss_l, ag_rs_l, rs_ss_l, rs_rs_l,
             left),
        )

        def win_start(b, off):
            s0 = jnp.clip(b * SQ_BLK + off - 128, 0, SKV - WIN)
            return pl.multiple_of(s0, 128)

        wins = ((kw_r, vw_r, kd_r, vd_r), (kw_l, vw_l, kd_l, vd_l))

        def win_dmas(ri, h):
            dr, off = rings[ri][0], rings[ri][1]
            kw, vw, kd, vd = wins[ri]
            b = lax.rem(my - dr * h + 2 * N_DEV, N_DEV)
            st = win_start(b, off)
            slot = h % 2
            cps = []
            for hh in range(HEADS):
                head = my * HEADS + hh
                cps.append(pltpu.make_async_copy(
                    k_hbm.at[pl.ds(st, WIN), head, :],
                    kw.at[slot, :, pl.ds(hh * DH, DH)], kd.at[slot]))
                cps.append(pltpu.make_async_copy(
                    v_hbm.at[pl.ds(st, WIN), head, :],
                    vw.at[slot, :, pl.ds(hh * DH, DH)], vd.at[slot]))
            return cps

        def compute_half(b, off, xf, kb, vb, start):
            xb = xf[pl.ds(b, 1)][0]
            q = jnp.dot(xb, wq_ref[...],
                        preferred_element_type=jnp.float32)
            qb = (q * SCALE).astype(jnp.bfloat16)
            ctx_parts = []
            for hh in range(HEADS):
                qh = qb[:, hh * DH:(hh + 1) * DH]
                kh = kb[:, hh * DH:(hh + 1) * DH]
                s = lax.dot_general(qh, kh, (((1,), (1,)), ((), ())),
                                    preferred_element_type=jnp.float32)
                qi = b * SQ_BLK + off + lax.broadcasted_iota(
                    jnp.int32, (RH, WIN), 0)
                ki = start + lax.broadcasted_iota(jnp.int32, (RH, WIN), 1)
                s = jnp.where(jnp.abs(qi - ki) <= 128, s, -1e9)
                m = jnp.max(s, axis=1, keepdims=True)
                w = jnp.exp(s - m)
                w = w / jnp.sum(w, axis=1, keepdims=True)
                vh = vb[:, hh * DH:(hh + 1) * DH]
                ctx_parts.append(jnp.dot(w.astype(jnp.bfloat16), vh,
                                         preferred_element_type=jnp.float32))
            ctx = jnp.concatenate(ctx_parts, axis=1).astype(jnp.bfloat16)
            partial = jnp.dot(ctx, wo_ref[...],
                              preferred_element_type=jnp.float32)
            return partial.astype(jnp.bfloat16)

        ag_sends = []
        rs_sends = []

        def rs_step(ring, s):
            dr, off, xf, pa, rr, ag_ss, ag_rs, rs_ss, rs_rs, tgt = ring
            bs = lax.rem(my - dr * (s + 1) + 2 * N_DEV, N_DEV)
            if s >= 1:
                pltpu.make_async_remote_copy(
                    src_ref=rr.at[s - 1], dst_ref=rr.at[s - 1],
                    send_sem=rs_ss.at[s - 1], recv_sem=rs_rs.at[s - 1],
                    device_id=(tgt,), device_id_type=pl.DeviceIdType.MESH,
                ).wait_recv()
                acc = (pa[pl.ds(bs, 1)][0].astype(jnp.float32)
                       + rr[s - 1].astype(jnp.float32))
                pa[pl.ds(bs, 1)] = acc.astype(jnp.bfloat16)[None]
            rdma = pltpu.make_async_remote_copy(
                src_ref=pa.at[bs], dst_ref=rr.at[s],
                send_sem=rs_ss.at[s], recv_sem=rs_rs.at[s],
                device_id=(tgt,), device_id_type=pl.DeviceIdType.MESH,
            )
            rdma.start()
            rs_sends.append(rdma)

        for ri in range(2):
            for c in win_dmas(ri, 0):
                c.start()

        for h in range(N_DEV):
            if h < N_DEV - 1:
                for ring in rings:
                    dr, off, xf = ring[0], ring[1], ring[2]
                    ag_ss, ag_rs, tgt = ring[5], ring[6], ring[9]
                    b = lax.rem(my - dr * h + 2 * N_DEV, N_DEV)
                    rdma = pltpu.make_async_remote_copy(
                        src_ref=xf.at[b], dst_ref=xf.at[b],
                        send_sem=ag_ss.at[h], recv_sem=ag_rs.at[h],
                        device_id=(tgt,),
                        device_id_type=pl.DeviceIdType.MESH,
                    )
                    rdma.start()
                    ag_sends.append(rdma)

                for ri in range(2):
                    for c in win_dmas(ri, h + 1):
                        c.start()

            if h >= 2:
                for ring in rings:
                    rs_step(ring, h - 2)

            for ri, (dr, off, xf, pa, rr, ag_ss, ag_rs, rs_ss, rs_rs,
                     tgt) in enumerate(rings):
                b = lax.rem(my - dr * h + 2 * N_DEV, N_DEV)
                for c in win_dmas(ri, h):
                    c.wait()
                kw, vw = wins[ri][0], wins[ri][1]
                kb = kw[h % 2].astype(jnp.bfloat16)
                vb = vw[h % 2].astype(jnp.bfloat16)
                chunk = compute_half(b, off, xf, kb, vb, win_start(b, off))
                pa[pl.ds(b, 1)] = chunk[None]

            if h < N_DEV - 1:
                for (dr, off, xf, pa, rr, ag_ss, ag_rs, rs_ss, rs_rs,
                     tgt) in rings:
                    b_next = lax.rem(my - dr * (h + 1) + 2 * N_DEV, N_DEV)
                    pltpu.make_async_remote_copy(
                        src_ref=xf.at[b_next], dst_ref=xf.at[b_next],
                        send_sem=ag_ss.at[h], recv_sem=ag_rs.at[h],
                        device_id=(tgt,),
                        device_id_type=pl.DeviceIdType.MESH,
                    ).wait_recv()

        for ring in rings:
            rs_step(ring, N_DEV - 2)

        outs = []
        for dr, off, xf, pa, rr, ag_ss, ag_rs, rs_ss, rs_rs, tgt in rings:
            pltpu.make_async_remote_copy(
                src_ref=rr.at[N_DEV - 2], dst_ref=rr.at[N_DEV - 2],
                send_sem=rs_ss.at[N_DEV - 2], recv_sem=rs_rs.at[N_DEV - 2],
                device_id=(tgt,), device_id_type=pl.DeviceIdType.MESH,
            ).wait_recv()
            outs.append((pa[pl.ds(my, 1)][0].astype(jnp.float32)
                         + rr[N_DEV - 2].astype(jnp.float32)))
        out_ref[0] = jnp.concatenate(outs, axis=0)

        for rdma in ag_sends + rs_sends:
            rdma.wait_send()

    out = pl.pallas_call(
        body,
        out_shape=jax.ShapeDtypeStruct((1, SQ_BLK, D_MODEL), jnp.float32),
        in_specs=[
            pl.BlockSpec(memory_space=pltpu.VMEM),
            pl.BlockSpec(memory_space=pltpu.VMEM),
            pl.BlockSpec(memory_space=pl.ANY),
            pl.BlockSpec(memory_space=pl.ANY),
            pl.BlockSpec(memory_space=pltpu.VMEM),
        ],
        out_specs=pl.BlockSpec(memory_space=pltpu.VMEM),
        scratch_shapes=[
            pltpu.VMEM((N_DEV, RH, D_MODEL), jnp.bfloat16),
            pltpu.VMEM((N_DEV, RH, D_MODEL), jnp.bfloat16),
            pltpu.VMEM((N_DEV, RH, D_MODEL), jnp.bfloat16),
            pltpu.VMEM((N_DEV, RH, D_MODEL), jnp.bfloat16),
            pltpu.VMEM((N_DEV - 1, RH, D_MODEL), jnp.bfloat16),
            pltpu.VMEM((N_DEV - 1, RH, D_MODEL), jnp.bfloat16),
            pltpu.VMEM((2, WIN, HEADS * DH), jnp.float32),
            pltpu.VMEM((2, WIN, HEADS * DH), jnp.float32),
            pltpu.VMEM((2, WIN, HEADS * DH), jnp.float32),
            pltpu.VMEM((2, WIN, HEADS * DH), jnp.float32),
        ] + [pltpu.SemaphoreType.DMA((N_DEV - 1,))] * 8
          + [pltpu.SemaphoreType.DMA((2,))] * 4,
        compiler_params=pltpu.CompilerParams(collective_id=0),
    )(x_bf, wq_bf, k2, v2, wo_bf)
    return out


# device time: 71544 ns/iter; 3.8074x vs baseline; 1.1305x over previous
import jax
import jax.numpy as jnp
from jax import lax
from jax.experimental import pallas as pl
from jax.experimental.pallas import tpu as pltpu

N_DEV = 8
SQ_BLK = 256
RH = 128
D_MODEL = 1024
HEADS = 8
DH = 128
SKV = 4096
WIN = 384
SCALE = 0.08838834764831843


def kernel(x, Wq, K_ext, V_ext, Wo):
    d = lax.axis_index("i")
    hs = d * HEADS

    x_bf = x[0].astype(jnp.bfloat16)
    wq_bf = Wq.astype(jnp.bfloat16)
    wo_bf = Wo.astype(jnp.bfloat16)
    k2 = K_ext[0]
    v2 = V_ext[0]

    def body(x_ref, wq_ref, k_hbm, v_hbm, wo_ref, out_ref,
             xf_r, xf_l, pa_r, pa_l, rr_r, rr_l,
             kw_r, vw_r, kw_l, vw_l,
             ag_ss_r, ag_rs_r, rs_ss_r, rs_rs_r,
             ag_ss_l, ag_rs_l, rs_ss_l, rs_rs_l,
             kd_r, vd_r, kd_l, vd_l):
        my = lax.axis_index("i")
        left = lax.rem(my + N_DEV - 1, N_DEV)
        right = lax.rem(my + 1, N_DEV)

        xf_r[pl.ds(my, 1)] = x_ref[:RH][None]
        xf_l[pl.ds(my, 1)] = x_ref[RH:][None]

        rings = (
            (+1, 0, xf_r, pa_r, rr_r, ag_ss_r, ag_rs_r, rs_ss_r, rs_rs_r,
             right),
            (-1, RH, xf_l, pa_l, rr_l, ag_ss_l, ag_rs_l, rs_ss_l, rs_rs_l,
             left),
        )

        def win_start(b, off):
            s0 = jnp.clip(b * SQ_BLK + off - 128, 0, SKV - WIN)
            return pl.multiple_of(s0, 128)

        wins = ((kw_r, vw_r, kd_r, vd_r), (kw_l, vw_l, kd_l, vd_l))

        def win_dmas(ri, h):
            dr, off = rings[ri][0], rings[ri][1]
            kw, vw, kd, vd = wins[ri]
            b = lax.rem(my - dr * h + 2 * N_DEV, N_DEV)
            st = win_start(b, off)
            slot = h % 2
            cps = []
            for hh in range(HEADS):
                head = my * HEADS + hh
                cps.append(pltpu.make_async_copy(
                    k_hbm.at[pl.ds(st, WIN), head, :],
                    kw.at[slot, :, pl.ds(hh * DH, DH)], kd.at[slot]))
                cps.append(pltpu.make_async_copy(
                    v_hbm.at[pl.ds(st, WIN), head, :],
                    vw.at[slot, :, pl.ds(hh * DH, DH)], vd.at[slot]))
            return cps

        def compute_half(b, off, xf, kb, vb, start):
            xb = xf[pl.ds(b, 1)][0]
            q = jnp.dot(xb, wq_ref[...],
                        preferred_element_type=jnp.float32)
            qb = (q * SCALE).astype(jnp.bfloat16)
            qi = b * SQ_BLK + off + lax.broadcasted_iota(
                jnp.int32, (RH, WIN), 0)
            ki = start + lax.broadcasted_iota(jnp.int32, (RH, WIN), 1)
            neg = jnp.where(jnp.abs(qi - ki) <= 128,
                            jnp.float32(0), jnp.float32(-1e9))
            ctx_parts = []
            for hh in range(HEADS):
                qh = qb[:, hh * DH:(hh + 1) * DH]
                kh = kb[:, hh * DH:(hh + 1) * DH]
                s = lax.dot_general(qh, kh, (((1,), (1,)), ((), ())),
                                    preferred_element_type=jnp.float32)
                s = s + neg
                m = jnp.max(s, axis=1, keepdims=True)
                w = jnp.exp(s - m)
                rcp = 1.0 / jnp.sum(w, axis=1, keepdims=True)
                vh = vb[:, hh * DH:(hh + 1) * DH]
                ctx_h = jnp.dot(w.astype(jnp.bfloat16), vh,
                                preferred_element_type=jnp.float32)
                ctx_parts.append(ctx_h * rcp)
            ctx = jnp.concatenate(ctx_parts, axis=1).astype(jnp.bfloat16)
            partial = jnp.dot(ctx, wo_ref[...],
                              preferred_element_type=jnp.float32)
            return partial.astype(jnp.bfloat16)

        ag_sends = []
        rs_sends = []

        def rs_step(ring, s):
            dr, off, xf, pa, rr, ag_ss, ag_rs, rs_ss, rs_rs, tgt = ring
            bs = lax.rem(my - dr * (s + 1) + 2 * N_DEV, N_DEV)
            if s >= 1:
                pltpu.make_async_remote_copy(
                    src_ref=rr.at[s - 1], dst_ref=rr.at[s - 1],
                    send_sem=rs_ss.at[s - 1], recv_sem=rs_rs.at[s - 1],
                    device_id=(tgt,), device_id_type=pl.DeviceIdType.MESH,
                ).wait_recv()
                acc = (pa[pl.ds(bs, 1)][0].astype(jnp.float32)
                       + rr[s - 1].astype(jnp.float32))
                pa[pl.ds(bs, 1)] = acc.astype(jnp.bfloat16)[None]
            rdma = pltpu.make_async_remote_copy(
                src_ref=pa.at[bs], dst_ref=rr.at[s],
                send_sem=rs_ss.at[s], recv_sem=rs_rs.at[s],
                device_id=(tgt,), device_id_type=pl.DeviceIdType.MESH,
            )
            rdma.start()
            rs_sends.append(rdma)

        for ri in range(2):
            for c in win_dmas(ri, 0):
                c.start()

        barrier = pltpu.get_barrier_semaphore()
        for nbr in (left, right):
            pl.semaphore_signal(barrier, inc=1, device_id=(nbr,),
                                device_id_type=pl.DeviceIdType.MESH)
        pl.semaphore_wait(barrier, 2)

        for h in range(N_DEV):
            if h < N_DEV - 1:
                for ring in rings:
                    dr, off, xf = ring[0], ring[1], ring[2]
                    ag_ss, ag_rs, tgt = ring[5], ring[6], ring[9]
                    b = lax.rem(my - dr * h + 2 * N_DEV, N_DEV)
                    rdma = pltpu.make_async_remote_copy(
                        src_ref=xf.at[b], dst_ref=xf.at[b],
                        send_sem=ag_ss.at[h], recv_sem=ag_rs.at[h],
                        device_id=(tgt,),
                        device_id_type=pl.DeviceIdType.MESH,
                    )
                    rdma.start()
                    ag_sends.append(rdma)

                for ri in range(2):
                    for c in win_dmas(ri, h + 1):
                        c.start()

            if h >= 2:
                for ring in rings:
                    rs_step(ring, h - 2)

            for ri, (dr, off, xf, pa, rr, ag_ss, ag_rs, rs_ss, rs_rs,
                     tgt) in enumerate(rings):
                b = lax.rem(my - dr * h + 2 * N_DEV, N_DEV)
                for c in win_dmas(ri, h):
                    c.wait()
                kw, vw = wins[ri][0], wins[ri][1]
                kb = kw[h % 2].astype(jnp.bfloat16)
                vb = vw[h % 2].astype(jnp.bfloat16)
                chunk = compute_half(b, off, xf, kb, vb, win_start(b, off))
                pa[pl.ds(b, 1)] = chunk[None]

            if h < N_DEV - 1:
                for (dr, off, xf, pa, rr, ag_ss, ag_rs, rs_ss, rs_rs,
                     tgt) in rings:
                    b_next = lax.rem(my - dr * (h + 1) + 2 * N_DEV, N_DEV)
                    pltpu.make_async_remote_copy(
                        src_ref=xf.at[b_next], dst_ref=xf.at[b_next],
                        send_sem=ag_ss.at[h], recv_sem=ag_rs.at[h],
                        device_id=(tgt,),
                        device_id_type=pl.DeviceIdType.MESH,
                    ).wait_recv()

        for ring in rings:
            rs_step(ring, N_DEV - 2)

        outs = []
        for dr, off, xf, pa, rr, ag_ss, ag_rs, rs_ss, rs_rs, tgt in rings:
            pltpu.make_async_remote_copy(
                src_ref=rr.at[N_DEV - 2], dst_ref=rr.at[N_DEV - 2],
                send_sem=rs_ss.at[N_DEV - 2], recv_sem=rs_rs.at[N_DEV - 2],
                device_id=(tgt,), device_id_type=pl.DeviceIdType.MESH,
            ).wait_recv()
            outs.append((pa[pl.ds(my, 1)][0].astype(jnp.float32)
                         + rr[N_DEV - 2].astype(jnp.float32)))
        out_ref[0] = jnp.concatenate(outs, axis=0)

        for rdma in ag_sends + rs_sends:
            rdma.wait_send()

    out = pl.pallas_call(
        body,
        out_shape=jax.ShapeDtypeStruct((1, SQ_BLK, D_MODEL), jnp.float32),
        in_specs=[
            pl.BlockSpec(memory_space=pltpu.VMEM),
            pl.BlockSpec(memory_space=pltpu.VMEM),
            pl.BlockSpec(memory_space=pl.ANY),
            pl.BlockSpec(memory_space=pl.ANY),
            pl.BlockSpec(memory_space=pltpu.VMEM),
        ],
        out_specs=pl.BlockSpec(memory_space=pltpu.VMEM),
        scratch_shapes=[
            pltpu.VMEM((N_DEV, RH, D_MODEL), jnp.bfloat16),
            pltpu.VMEM((N_DEV, RH, D_MODEL), jnp.bfloat16),
            pltpu.VMEM((N_DEV, RH, D_MODEL), jnp.bfloat16),
            pltpu.VMEM((N_DEV, RH, D_MODEL), jnp.bfloat16),
            pltpu.VMEM((N_DEV - 1, RH, D_MODEL), jnp.bfloat16),
            pltpu.VMEM((N_DEV - 1, RH, D_MODEL), jnp.bfloat16),
            pltpu.VMEM((2, WIN, HEADS * DH), jnp.float32),
            pltpu.VMEM((2, WIN, HEADS * DH), jnp.float32),
            pltpu.VMEM((2, WIN, HEADS * DH), jnp.float32),
            pltpu.VMEM((2, WIN, HEADS * DH), jnp.float32),
        ] + [pltpu.SemaphoreType.DMA((N_DEV - 1,))] * 8
          + [pltpu.SemaphoreType.DMA((2,))] * 4,
        compiler_params=pltpu.CompilerParams(collective_id=0),
    )(x_bf, wq_bf, k2, v2, wo_bf)
    return out


# device time: 64950 ns/iter; 4.1940x vs baseline; 1.1015x over previous
import jax
import jax.numpy as jnp
from jax import lax
from jax.experimental import pallas as pl
from jax.experimental.pallas import tpu as pltpu

N_DEV = 8
SQ_BLK = 256
RH = 128
D_MODEL = 1024
HEADS = 8
DH = 128
SKV = 4096
WIN = 384
SCALE = 0.08838834764831843


def kernel(x, Wq, K_ext, V_ext, Wo):
    d = lax.axis_index("i")
    hs = d * HEADS

    x_bf = x[0].astype(jnp.bfloat16)
    wq_bf = (Wq * SCALE).astype(jnp.bfloat16)
    wo_bf = Wo.astype(jnp.bfloat16)
    k2 = K_ext[0]
    v2 = V_ext[0]

    def body(x_ref, wq_ref, k_hbm, v_hbm, wo_ref, out_ref,
             xf_r, xf_l, pa_r, pa_l, rr_r, rr_l,
             kw_r, vw_r, kw_l, vw_l,
             ag_ss_r, ag_rs_r, rs_ss_r, rs_rs_r,
             ag_ss_l, ag_rs_l, rs_ss_l, rs_rs_l,
             kd_r, vd_r, kd_l, vd_l):
        my = lax.axis_index("i")
        left = lax.rem(my + N_DEV - 1, N_DEV)
        right = lax.rem(my + 1, N_DEV)

        xf_r[pl.ds(my, 1)] = x_ref[:RH][None]
        xf_l[pl.ds(my, 1)] = x_ref[RH:][None]

        rings = (
            (+1, 0, xf_r, pa_r, rr_r, ag_ss_r, ag_rs_r, rs_ss_r, rs_rs_r,
             right),
            (-1, RH, xf_l, pa_l, rr_l, ag_ss_l, ag_rs_l, rs_ss_l, rs_rs_l,
             left),
        )

        def win_start(b, off):
            s0 = jnp.clip(b * SQ_BLK + off - 128, 0, SKV - WIN)
            return pl.multiple_of(s0, 128)

        wins = ((kw_r, vw_r, kd_r, vd_r), (kw_l, vw_l, kd_l, vd_l))

        def win_dmas(ri, h):
            dr, off = rings[ri][0], rings[ri][1]
            kw, vw, kd, vd = wins[ri]
            b = lax.rem(my - dr * h + 2 * N_DEV, N_DEV)
            st = win_start(b, off)
            slot = h % 2
            cps = []
            for hh in range(HEADS):
                head = my * HEADS + hh
                cps.append(pltpu.make_async_copy(
                    k_hbm.at[pl.ds(st, WIN), head, :],
                    kw.at[slot, :, pl.ds(hh * DH, DH)], kd.at[slot]))
                cps.append(pltpu.make_async_copy(
                    v_hbm.at[pl.ds(st, WIN), head, :],
                    vw.at[slot, :, pl.ds(hh * DH, DH)], vd.at[slot]))
            return cps

        def compute_half(b, off, xf, kb, vb, start):
            xb = xf[pl.ds(b, 1)][0]
            qb = jnp.dot(xb, wq_ref[...],
                         preferred_element_type=jnp.float32
                         ).astype(jnp.bfloat16)
            qi = b * SQ_BLK + off + lax.broadcasted_iota(
                jnp.int32, (RH, WIN), 0)
            ki = start + lax.broadcasted_iota(jnp.int32, (RH, WIN), 1)
            neg = jnp.where(jnp.abs(qi - ki) <= 128,
                            jnp.float32(0), jnp.float32(-1e9))
            ctx_parts = []
            for hh in range(HEADS):
                qh = qb[:, hh * DH:(hh + 1) * DH]
                kh = kb[:, hh * DH:(hh + 1) * DH]
                s = lax.dot_general(qh, kh, (((1,), (1,)), ((), ())),
                                    preferred_element_type=jnp.float32)
                w = jnp.exp(s + neg)
                rcp = 1.0 / jnp.sum(w, axis=1, keepdims=True)
                vh = vb[:, hh * DH:(hh + 1) * DH]
                ctx_h = jnp.dot(w.astype(jnp.bfloat16), vh,
                                preferred_element_type=jnp.float32)
                ctx_parts.append(ctx_h * rcp)
            ctx = jnp.concatenate(ctx_parts, axis=1).astype(jnp.bfloat16)
            partial = jnp.dot(ctx, wo_ref[...],
                              preferred_element_type=jnp.float32)
            return partial.astype(jnp.bfloat16)

        ag_sends = []
        rs_sends = []

        def rs_step(ring, s):
            dr, off, xf, pa, rr, ag_ss, ag_rs, rs_ss, rs_rs, tgt = ring
            bs = lax.rem(my - dr * (s + 1) + 2 * N_DEV, N_DEV)
            if s >= 1:
                pltpu.make_async_remote_copy(
                    src_ref=rr.at[s - 1], dst_ref=rr.at[s - 1],
                    send_sem=rs_ss.at[s - 1], recv_sem=rs_rs.at[s - 1],
                    device_id=(tgt,), device_id_type=pl.DeviceIdType.MESH,
                ).wait_recv()
                acc = (pa[pl.ds(bs, 1)][0].astype(jnp.float32)
                       + rr[s - 1].astype(jnp.float32))
                pa[pl.ds(bs, 1)] = acc.astype(jnp.bfloat16)[None]
            rdma = pltpu.make_async_remote_copy(
                src_ref=pa.at[bs], dst_ref=rr.at[s],
                send_sem=rs_ss.at[s], recv_sem=rs_rs.at[s],
                device_id=(tgt,), device_id_type=pl.DeviceIdType.MESH,
            )
            rdma.start()
            rs_sends.append(rdma)

        for ri in range(2):
            for c in win_dmas(ri, 0):
                c.start()

        barrier = pltpu.get_barrier_semaphore()
        for nbr in (left, right):
            pl.semaphore_signal(barrier, inc=1, device_id=(nbr,),
                                device_id_type=pl.DeviceIdType.MESH)
        pl.semaphore_wait(barrier, 2)

        for h in range(N_DEV):
            if h < N_DEV - 1:
                for ring in rings:
                    dr, off, xf = ring[0], ring[1], ring[2]
                    ag_ss, ag_rs, tgt = ring[5], ring[6], ring[9]
                    b = lax.rem(my - dr * h + 2 * N_DEV, N_DEV)
                    rdma = pltpu.make_async_remote_copy(
                        src_ref=xf.at[b], dst_ref=xf.at[b],
                        send_sem=ag_ss.at[h], recv_sem=ag_rs.at[h],
                        device_id=(tgt,),
                        device_id_type=pl.DeviceIdType.MESH,
                    )
                    rdma.start()
                    ag_sends.append(rdma)

                for ri in range(2):
                    for c in win_dmas(ri, h + 1):
                        c.start()

            if h >= 2:
                for ring in rings:
                    rs_step(ring, h - 2)

            for ri, (dr, off, xf, pa, rr, ag_ss, ag_rs, rs_ss, rs_rs,
                     tgt) in enumerate(rings):
                b = lax.rem(my - dr * h + 2 * N_DEV, N_DEV)
                for c in win_dmas(ri, h):
                    c.wait()
                kw, vw = wins[ri][0], wins[ri][1]
                kb = kw[h % 2].astype(jnp.bfloat16)
                vb = vw[h % 2].astype(jnp.bfloat16)
                chunk = compute_half(b, off, xf, kb, vb, win_start(b, off))
                pa[pl.ds(b, 1)] = chunk[None]

            if h < N_DEV - 1:
                for (dr, off, xf, pa, rr, ag_ss, ag_rs, rs_ss, rs_rs,
                     tgt) in rings:
                    b_next = lax.rem(my - dr * (h + 1) + 2 * N_DEV, N_DEV)
                    pltpu.make_async_remote_copy(
                        src_ref=xf.at[b_next], dst_ref=xf.at[b_next],
                        send_sem=ag_ss.at[h], recv_sem=ag_rs.at[h],
                        device_id=(tgt,),
                        device_id_type=pl.DeviceIdType.MESH,
                    ).wait_recv()

        for ring in rings:
            rs_step(ring, N_DEV - 2)

        outs = []
        for dr, off, xf, pa, rr, ag_ss, ag_rs, rs_ss, rs_rs, tgt in rings:
            pltpu.make_async_remote_copy(
                src_ref=rr.at[N_DEV - 2], dst_ref=rr.at[N_DEV - 2],
                send_sem=rs_ss.at[N_DEV - 2], recv_sem=rs_rs.at[N_DEV - 2],
                device_id=(tgt,), device_id_type=pl.DeviceIdType.MESH,
            ).wait_recv()
            outs.append((pa[pl.ds(my, 1)][0].astype(jnp.float32)
                         + rr[N_DEV - 2].astype(jnp.float32)))
        out_ref[0] = jnp.concatenate(outs, axis=0)

        for rdma in ag_sends + rs_sends:
            rdma.wait_send()

    out = pl.pallas_call(
        body,
        out_shape=jax.ShapeDtypeStruct((1, SQ_BLK, D_MODEL), jnp.float32),
        in_specs=[
            pl.BlockSpec(memory_space=pltpu.VMEM),
            pl.BlockSpec(memory_space=pltpu.VMEM),
            pl.BlockSpec(memory_space=pl.ANY),
            pl.BlockSpec(memory_space=pl.ANY),
            pl.BlockSpec(memory_space=pltpu.VMEM),
        ],
        out_specs=pl.BlockSpec(memory_space=pltpu.VMEM),
        scratch_shapes=[
            pltpu.VMEM((N_DEV, RH, D_MODEL), jnp.bfloat16),
            pltpu.VMEM((N_DEV, RH, D_MODEL), jnp.bfloat16),
            pltpu.VMEM((N_DEV, RH, D_MODEL), jnp.bfloat16),
            pltpu.VMEM((N_DEV, RH, D_MODEL), jnp.bfloat16),
            pltpu.VMEM((N_DEV - 1, RH, D_MODEL), jnp.bfloat16),
            pltpu.VMEM((N_DEV - 1, RH, D_MODEL), jnp.bfloat16),
            pltpu.VMEM((2, WIN, HEADS * DH), jnp.float32),
            pltpu.VMEM((2, WIN, HEADS * DH), jnp.float32),
            pltpu.VMEM((2, WIN, HEADS * DH), jnp.float32),
            pltpu.VMEM((2, WIN, HEADS * DH), jnp.float32),
        ] + [pltpu.SemaphoreType.DMA((N_DEV - 1,))] * 8
          + [pltpu.SemaphoreType.DMA((2,))] * 4,
        compiler_params=pltpu.CompilerParams(collective_id=0),
    )(x_bf, wq_bf, k2, v2, wo_bf)
    return out
